# Initial kernel scaffold; baseline (speedup 1.0000x reference)
import jax
import jax.numpy as jnp
from jax.experimental import pallas as pl


def kernel(atom_type_indices, embedding_table):
    raise NotImplementedError("write your pallas kernel here")


# speedup vs baseline: 6.5040x; 6.5040x over previous
import functools

import jax
import jax.numpy as jnp
from jax import lax
from jax.experimental import pallas as pl
from jax.experimental.pallas import tpu as pltpu
from jax.experimental.pallas import tpu_sc as plsc

NUM_ROWS = 16384
SEQ = 200
EMBED = 32
B = NUM_ROWS * SEQ
NC = 2
NS = 16
NW = NC * NS
B_PER_W = B // NW
CHUNK = 1024
N_CHUNKS = B_PER_W // CHUNK

assert B % NW == 0 and B_PER_W % CHUNK == 0 and N_CHUNKS >= 4

_mesh = plsc.VectorSubcoreMesh(core_axis_name="c", subcore_axis_name="s")


@functools.partial(
    pl.kernel,
    mesh=_mesh,
    out_type=jax.ShapeDtypeStruct((B, EMBED), jnp.float32),
    compiler_params=pltpu.CompilerParams(use_tc_tiling_on_sc=False),
    scratch_types=[
        pltpu.VMEM((CHUNK,), jnp.int32),
        pltpu.VMEM((CHUNK,), jnp.int32),
        pltpu.VMEM((CHUNK, EMBED), jnp.float32),
        pltpu.VMEM((CHUNK, EMBED), jnp.float32),
        pltpu.SemaphoreType.DMA,
        pltpu.SemaphoreType.DMA,
        pltpu.SemaphoreType.DMA,
        pltpu.SemaphoreType.DMA,
        pltpu.SemaphoreType.DMA,
        pltpu.SemaphoreType.DMA,
    ],
)
def _gather(idx_hbm, table_hbm, out_hbm,
            idx0, idx1, rows0, rows1,
            isem0, isem1, gsem0, gsem1, osem0, osem1):
    wid = lax.axis_index("s") * NC + lax.axis_index("c")
    base = wid * B_PER_W

    idx_v = (idx0, idx1)
    rows_v = (rows0, rows1)
    isem = (isem0, isem1)
    gsem = (gsem0, gsem1)
    osem = (osem0, osem1)

    def start_idx(c, b):
        pltpu.async_copy(idx_hbm.at[pl.ds(base + c * CHUNK, CHUNK)],
                         idx_v[b], isem[b])

    def wait_idx(c, b):
        pltpu.make_async_copy(idx_hbm.at[pl.ds(base + c * CHUNK, CHUNK)],
                              idx_v[b], isem[b]).wait()

    def start_gather(b):
        pltpu.async_copy(table_hbm.at[idx_v[b]], rows_v[b], gsem[b])

    def wait_gather(b):
        pltpu.make_async_copy(table_hbm.at[idx_v[b]], rows_v[b],
                              gsem[b]).wait()

    def start_out(c, b):
        pltpu.async_copy(rows_v[b],
                         out_hbm.at[pl.ds(base + c * CHUNK, CHUNK)], osem[b])

    def wait_out(c, b):
        pltpu.make_async_copy(rows_v[b],
                              out_hbm.at[pl.ds(base + c * CHUNK, CHUNK)],
                              osem[b]).wait()

    start_idx(0, 0)
    wait_idx(0, 0)
    start_gather(0)
    start_idx(1, 1)
    wait_idx(1, 1)
    start_gather(1)
    wait_gather(0)
    start_idx(2, 0)
    start_out(0, 0)

    def pair(p, carry):
        i0 = 2 * p + 2
        wait_idx(i0, 0)
        wait_out(i0 - 2, 0)
        start_gather(0)
        wait_gather(1)
        start_idx(i0 + 1, 1)
        start_out(i0 - 1, 1)
        i1 = i0 + 1
        wait_idx(i1, 1)
        wait_out(i1 - 2, 1)
        start_gather(1)
        wait_gather(0)
        start_idx(i1 + 1, 0)
        start_out(i1 - 1, 0)
        return carry

    lax.fori_loop(0, (N_CHUNKS - 4) // 2, pair, 0, unroll=False)

    n2, n1 = N_CHUNKS - 2, N_CHUNKS - 1
    wait_idx(n2, 0)
    wait_out(n2 - 2, 0)
    start_gather(0)
    wait_gather(1)
    start_idx(n1, 1)
    start_out(n2 - 1, 1)
    wait_idx(n1, 1)
    wait_out(n1 - 2, 1)
    start_gather(1)
    wait_gather(0)
    start_out(n2, 0)
    wait_gather(1)
    start_out(n1, 1)
    wait_out(n2, 0)
    wait_out(n1, 1)


def kernel(atom_type_indices, embedding_table):
    idx_flat = atom_type_indices.reshape(B)
    out = _gather(idx_flat, embedding_table)
    return out.reshape(NUM_ROWS, SEQ, EMBED)
